# baseline (speedup 1.0000x reference)
import functools

import jax
import jax.numpy as jnp
from jax import lax
from jax.experimental import pallas as pl
from jax.experimental.pallas import tpu as pltpu
from jax.experimental.pallas import tpu_sc as plsc

N = 10000
E = 160000
EPS = 1e-5
B = 128
EP = 163840
NBLK = EP // (16 * B)
NBLK_D = EP // (32 * B)
NACC = 10240
BN = 1000
GRID = N // BN
F32 = jnp.float32

_mesh = plsc.VectorSubcoreMesh(core_axis_name="c", subcore_axis_name="s")
_sc_params = pltpu.CompilerParams(use_tc_tiling_on_sc=False)
_tc_params = pltpu.CompilerParams(vmem_limit_bytes=100 * 1024 * 1024)


@functools.partial(
    pl.kernel,
    out_type=jax.ShapeDtypeStruct((2, NACC, 16), F32),
    mesh=_mesh,
    compiler_params=_sc_params,
    scratch_types=[
        pltpu.VMEM((NBLK_D, B), jnp.int32),
        pltpu.VMEM((B, 16), F32),
        pltpu.VMEM((B, 16), F32),
        pltpu.VMEM_SHARED((NACC, 16), F32),
    ],
)
def _deg_kernel(dst3, out, dst_v, ones_v, zero_v, acc):
    cid = lax.axis_index("c")
    tid = lax.axis_index("s")
    pltpu.sync_copy(dst3.at[tid, pl.ds(cid * NBLK_D, NBLK_D)], dst_v)

    def fill(i, _):
        ones_v[i, pl.ds(0, 16)] = jnp.full((16,), 1.0, F32)
        zero_v[i, pl.ds(0, 16)] = jnp.zeros((16,), F32)
        return 0

    lax.fori_loop(0, B, fill, 0)
    rz = NACC // 16
    for z in range(rz // B):
        pltpu.sync_copy(zero_v, acc.at[pl.ds(tid * rz + z * B, B)])
    plsc.subcore_barrier()

    def step(j, _):
        pltpu.sync_copy(ones_v, acc.at[dst_v.at[j]], add=True)
        return 0

    lax.fori_loop(0, NBLK_D, step, 0)
    plsc.subcore_barrier()
    rpt = NACC // 16
    pltpu.sync_copy(acc.at[pl.ds(tid * rpt, rpt)],
                    out.at[cid, pl.ds(tid * rpt, rpt)])


def _make_agg(C, W, pipelined=True):
    wpr = W // 16

    @functools.partial(
        pl.kernel,
        out_type=jax.ShapeDtypeStruct((C * NACC, W), F32),
        mesh=_mesh,
        compiler_params=_sc_params,
        scratch_types=[
            pltpu.VMEM((NBLK, B), jnp.int32),
            pltpu.VMEM((NBLK, B), jnp.int32),
            pltpu.VMEM((B, W), F32),
            pltpu.VMEM((B, W), F32),
            pltpu.VMEM((B, W), F32),
            pltpu.VMEM((B, W), F32),
            pltpu.VMEM((B, W), F32),
            pltpu.VMEM_SHARED((NACC, W), F32),
            pltpu.SemaphoreType.DMA,
            pltpu.SemaphoreType.DMA,
            pltpu.SemaphoreType.DMA,
            pltpu.SemaphoreType.DMA,
            pltpu.SemaphoreType.DMA,
        ],
    )
    def agg_kernel(srcadj, dst3, h, out, adj_v, dst_v, r0, r1, r2, r3, zv,
                   acc, g0, g1, g2, g3, ssc):
        bufs = (r0, r1, r2, r3)
        gsem = (g0, g1, g2, g3)
        cid = lax.axis_index("c")
        tid = lax.axis_index("s")
        pltpu.sync_copy(dst3.at[tid], dst_v)

        def zfill(i, _):
            zv[i // wpr, pl.ds((i % wpr) * 16, 16)] = jnp.zeros((16,), F32)
            return 0

        lax.fori_loop(0, B * wpr, zfill, 0)

        for kc in range(C // 2):
            chunk = kc * 2 + cid
            rz = NACC // 16
            for z in range(rz // B):
                pltpu.sync_copy(zv, acc.at[pl.ds(tid * rz + z * B, B)])

            pltpu.sync_copy(srcadj.at[chunk, tid], adj_v)
            plsc.subcore_barrier()

            if not pipelined:
                def sstep(j, _):
                    pltpu.async_copy(h.at[adj_v.at[j]], r0, g0).wait()
                    pltpu.sync_copy(r0, acc.at[dst_v.at[j]], add=True)
                    return 0

                lax.fori_loop(0, NBLK, sstep, 0)
                plsc.subcore_barrier()
                rpt = NACC // 16
                pltpu.sync_copy(acc.at[pl.ds(tid * rpt, rpt)],
                                out.at[pl.ds(chunk * NACC + tid * rpt, rpt)])
                plsc.subcore_barrier()
                continue

            for b in range(4):
                pltpu.async_copy(h.at[adj_v.at[b]], bufs[b], gsem[b])

            def step(i, _):
                for b in range(4):
                    j = 4 * i + b
                    pltpu.make_async_copy(h.at[adj_v.at[j]], bufs[b],
                                          gsem[b]).wait()
                    pltpu.sync_copy(bufs[b], acc.at[dst_v.at[j]], add=True)

                    @pl.when(j + 4 < NBLK)
                    def _(b=b, j=j):
                        pltpu.async_copy(h.at[adj_v.at[j + 4]], bufs[b],
                                         gsem[b])
                return 0

            lax.fori_loop(0, NBLK // 4, step, 0)
            plsc.subcore_barrier()
            rpt = NACC // 16
            pltpu.sync_copy(acc.at[pl.ds(tid * rpt, rpt)],
                            out.at[pl.ds(chunk * NACC + tid * rpt, rpt)])
            plsc.subcore_barrier()

    return agg_kernel


_agg_x = _make_agg(4, 64)
_agg_h = _make_agg(8, 64)
_agg_y = _make_agg(2, 16)


_DN = (((1,), (1,)), ((), ()))


def _make_tcA(C_in, W_in, dout):
    din = C_in * W_in
    C_out = dout // 64

    def body(agg_ref, degp_ref, h_ref, wl_ref, wr_ref, b_ref,
             z_ref, s_ref, q_ref):
        deg = degp_ref[0, :, 0:1] + degp_ref[1, :, 0:1]
        inv = 1.0 / jnp.maximum(deg, 1.0)
        z = jnp.zeros((BN, dout), F32)
        for c in range(C_in):
            a = agg_ref[c] * inv
            z = z + lax.dot_general(a, wl_ref[:, c * W_in:(c + 1) * W_in],
                                    _DN, preferred_element_type=F32)
            z = z + lax.dot_general(h_ref[c], wr_ref[:, c * W_in:(c + 1) * W_in],
                                    _DN, preferred_element_type=F32)
        z = z + b_ref[...]
        for c in range(C_out):
            z_ref[c] = z[:, c * 64:(c + 1) * 64]
        sp = jnp.sum(z, axis=0, keepdims=True)
        qp = jnp.sum(z * z, axis=0, keepdims=True)

        @pl.when(pl.program_id(0) == 0)
        def _():
            s_ref[...] = sp
            q_ref[...] = qp

        @pl.when(pl.program_id(0) > 0)
        def _():
            s_ref[...] += sp
            q_ref[...] += qp

    return pl.pallas_call(
        body,
        grid=(GRID,),
        compiler_params=_tc_params,
        in_specs=[
            pl.BlockSpec((C_in, BN, W_in), lambda i: (0, i, 0)),
            pl.BlockSpec((2, BN, 16), lambda i: (0, i, 0)),
            pl.BlockSpec((C_in, BN, W_in), lambda i: (0, i, 0)),
            pl.BlockSpec((dout, din), lambda i: (0, 0)),
            pl.BlockSpec((dout, din), lambda i: (0, 0)),
            pl.BlockSpec((1, dout), lambda i: (0, 0)),
        ],
        out_specs=[
            pl.BlockSpec((C_out, BN, 64), lambda i: (0, i, 0)),
            pl.BlockSpec((1, dout), lambda i: (0, 0)),
            pl.BlockSpec((1, dout), lambda i: (0, 0)),
        ],
        out_shape=[
            jax.ShapeDtypeStruct((C_out, N, 64), F32),
            jax.ShapeDtypeStruct((1, dout), F32),
            jax.ShapeDtypeStruct((1, dout), F32),
        ],
    )


_tcA0 = _make_tcA(4, 64, 512)
_tcA1 = _make_tcA(8, 64, 512)


def _bn_coefs(s_ref, q_ref, g_ref, be_ref):
    mu = s_ref[...] * (1.0 / N)
    var = q_ref[...] * (1.0 / N) - mu * mu
    scale = g_ref[...] * lax.rsqrt(var + EPS)
    shift = be_ref[...] - mu * scale
    return scale, shift


def _make_tcB():
    def body(z_ref, s_ref, q_ref, g_ref, be_ref, h_ref):
        scale, shift = _bn_coefs(s_ref, q_ref, g_ref, be_ref)
        for c in range(8):
            hz = (z_ref[c] * scale[:, c * 64:(c + 1) * 64]
                  + shift[:, c * 64:(c + 1) * 64])
            h_ref[c] = jnp.maximum(hz, 0.0)

    return pl.pallas_call(
        body,
        grid=(GRID,),
        compiler_params=_tc_params,
        in_specs=[
            pl.BlockSpec((8, BN, 64), lambda i: (0, i, 0)),
            pl.BlockSpec((1, 512), lambda i: (0, 0)),
            pl.BlockSpec((1, 512), lambda i: (0, 0)),
            pl.BlockSpec((1, 512), lambda i: (0, 0)),
            pl.BlockSpec((1, 512), lambda i: (0, 0)),
        ],
        out_specs=[pl.BlockSpec((8, BN, 64), lambda i: (0, i, 0))],
        out_shape=[jax.ShapeDtypeStruct((8, N, 64), F32)],
    )


_tcB = _make_tcB()


def _make_tcBp():
    def body(z_ref, s_ref, q_ref, g_ref, be_ref, wl2_ref, h_ref, y_ref):
        scale, shift = _bn_coefs(s_ref, q_ref, g_ref, be_ref)
        y = jnp.zeros((BN, 32), F32)
        for c in range(8):
            hz = (z_ref[c] * scale[:, c * 64:(c + 1) * 64]
                  + shift[:, c * 64:(c + 1) * 64])
            hc = jnp.maximum(hz, 0.0)
            h_ref[c] = hc
            y = y + lax.dot_general(hc, wl2_ref[:, c * 64:(c + 1) * 64],
                                    _DN, preferred_element_type=F32)
        y_ref[0] = y[:, 0:16]
        y_ref[1] = y[:, 16:32]

    return pl.pallas_call(
        body,
        grid=(GRID,),
        compiler_params=_tc_params,
        in_specs=[
            pl.BlockSpec((8, BN, 64), lambda i: (0, i, 0)),
            pl.BlockSpec((1, 512), lambda i: (0, 0)),
            pl.BlockSpec((1, 512), lambda i: (0, 0)),
            pl.BlockSpec((1, 512), lambda i: (0, 0)),
            pl.BlockSpec((1, 512), lambda i: (0, 0)),
            pl.BlockSpec((32, 512), lambda i: (0, 0)),
        ],
        out_specs=[
            pl.BlockSpec((8, BN, 64), lambda i: (0, i, 0)),
            pl.BlockSpec((2, BN, 16), lambda i: (0, i, 0)),
        ],
        out_shape=[
            jax.ShapeDtypeStruct((8, N, 64), F32),
            jax.ShapeDtypeStruct((2, N, 16), F32),
        ],
    )


_tcBp = _make_tcBp()


def _make_tcA2():
    def body(agg_ref, degp_ref, h_ref, wr_ref, b_ref, z_ref, s_ref, q_ref):
        deg = degp_ref[0, :, 0:1] + degp_ref[1, :, 0:1]
        inv = 1.0 / jnp.maximum(deg, 1.0)
        mean = jnp.concatenate([agg_ref[0], agg_ref[1]], axis=1) * inv
        z = mean + b_ref[...]
        for c in range(8):
            z = z + lax.dot_general(h_ref[c], wr_ref[:, c * 64:(c + 1) * 64],
                                    _DN, preferred_element_type=F32)
        z_ref[...] = z
        sp = jnp.sum(z, axis=0, keepdims=True)
        qp = jnp.sum(z * z, axis=0, keepdims=True)

        @pl.when(pl.program_id(0) == 0)
        def _():
            s_ref[...] = sp
            q_ref[...] = qp

        @pl.when(pl.program_id(0) > 0)
        def _():
            s_ref[...] += sp
            q_ref[...] += qp

    return pl.pallas_call(
        body,
        grid=(GRID,),
        compiler_params=_tc_params,
        in_specs=[
            pl.BlockSpec((2, BN, 16), lambda i: (0, i, 0)),
            pl.BlockSpec((2, BN, 16), lambda i: (0, i, 0)),
            pl.BlockSpec((8, BN, 64), lambda i: (0, i, 0)),
            pl.BlockSpec((32, 512), lambda i: (0, 0)),
            pl.BlockSpec((1, 32), lambda i: (0, 0)),
        ],
        out_specs=[
            pl.BlockSpec((BN, 32), lambda i: (i, 0)),
            pl.BlockSpec((1, 32), lambda i: (0, 0)),
            pl.BlockSpec((1, 32), lambda i: (0, 0)),
        ],
        out_shape=[
            jax.ShapeDtypeStruct((N, 32), F32),
            jax.ShapeDtypeStruct((1, 32), F32),
            jax.ShapeDtypeStruct((1, 32), F32),
        ],
    )


_tcA2 = _make_tcA2()


def _make_tcC():
    def body(z_ref, s_ref, q_ref, g_ref, be_ref, hw_ref, hb_ref, o_ref):
        scale, shift = _bn_coefs(s_ref, q_ref, g_ref, be_ref)
        h3 = jnp.maximum(z_ref[...] * scale + shift, 0.0)
        o_ref[...] = (jnp.sum(h3 * hw_ref[...], axis=1, keepdims=True)
                      + hb_ref[0, 0])

    return pl.pallas_call(
        body,
        grid=(GRID,),
        compiler_params=_tc_params,
        in_specs=[
            pl.BlockSpec((BN, 32), lambda i: (i, 0)),
            pl.BlockSpec((1, 32), lambda i: (0, 0)),
            pl.BlockSpec((1, 32), lambda i: (0, 0)),
            pl.BlockSpec((1, 32), lambda i: (0, 0)),
            pl.BlockSpec((1, 32), lambda i: (0, 0)),
            pl.BlockSpec((1, 32), lambda i: (0, 0)),
            pl.BlockSpec((1, 1), lambda i: (0, 0)),
        ],
        out_specs=[pl.BlockSpec((BN, 1), lambda i: (i, 0))],
        out_shape=[jax.ShapeDtypeStruct((N, 1), F32)],
    )


_tcC = _make_tcC()


def kernel(x, edge_index, Wl0, Wr0, b0, gamma0, beta0, Wl1, Wr1, b1, gamma1,
           beta1, Wl2, Wr2, b2, gamma2, beta2, headW, headb):
    src = edge_index[0]
    dst = edge_index[1]
    pad = EP - E
    src3 = jnp.concatenate([src, jnp.zeros((pad,), jnp.int32)]).reshape(
        16, NBLK, B)
    dst3 = jnp.concatenate([dst, jnp.full((pad,), N, jnp.int32)]).reshape(
        16, NBLK, B)

    def _srcadj(C):
        offs = (jnp.arange(C, dtype=jnp.int32) * N)[:, None, None, None]
        return src3[None] + offs
    x_cm = x.reshape(N, 4, 64).transpose(1, 0, 2)

    degp = _deg_kernel(dst3)

    agg0 = _agg_x(_srcadj(4), dst3, x_cm.reshape(4 * N, 64))
    z0, s0, q0 = _tcA0(agg0.reshape(4, NACC, 64), degp, x_cm, Wl0, Wr0,
                       b0.reshape(1, 512))
    (h1,) = _tcB(z0, s0, q0, gamma0.reshape(1, 512), beta0.reshape(1, 512))

    agg1 = _agg_h(_srcadj(8), dst3, h1.reshape(8 * N, 64))
    z1, s1, q1 = _tcA1(agg1.reshape(8, NACC, 64), degp, h1, Wl1, Wr1,
                       b1.reshape(1, 512))
    h2, y2 = _tcBp(z1, s1, q1, gamma1.reshape(1, 512), beta1.reshape(1, 512),
                   Wl2)

    agg2 = _agg_y(_srcadj(2), dst3, y2.reshape(2 * N, 16))
    z2, s2, q2 = _tcA2(agg2.reshape(2, NACC, 16), degp, h2, Wr2,
                       b2.reshape(1, 32))
    (out,) = _tcC(z2, s2, q2, gamma2.reshape(1, 32), beta2.reshape(1, 32),
                  headW, headb.reshape(1, 1))
    return out


# speedup vs baseline: 1.0741x; 1.0741x over previous
import functools

import jax
import jax.numpy as jnp
from jax import lax
from jax.experimental import pallas as pl
from jax.experimental.pallas import tpu as pltpu
from jax.experimental.pallas import tpu_sc as plsc

N = 10000
E = 160000
EPS = 1e-5
B = 128
EP = 163840
NBLK = EP // (16 * B)
NBLK_D = EP // (32 * B)
NACC = 10240
BN = 1000
GRID = N // BN
F32 = jnp.float32

_mesh = plsc.VectorSubcoreMesh(core_axis_name="c", subcore_axis_name="s")
_sc_params = pltpu.CompilerParams(use_tc_tiling_on_sc=False)


@functools.partial(
    pl.kernel,
    out_type=jax.ShapeDtypeStruct((2, NACC, 16), F32),
    mesh=_mesh,
    compiler_params=_sc_params,
    scratch_types=[
        pltpu.VMEM((NBLK_D, B), jnp.int32),
        pltpu.VMEM((B, 16), F32),
        pltpu.VMEM((B, 16), F32),
        pltpu.VMEM_SHARED((NACC, 16), F32),
    ],
)
def _deg_kernel(dst3, out, dst_v, ones_v, zero_v, acc):
    cid = lax.axis_index("c")
    tid = lax.axis_index("s")
    pltpu.sync_copy(dst3.at[tid, pl.ds(cid * NBLK_D, NBLK_D)], dst_v)

    def fill(i, _):
        ones_v[i, pl.ds(0, 16)] = jnp.full((16,), 1.0, F32)
        zero_v[i, pl.ds(0, 16)] = jnp.zeros((16,), F32)
        return 0

    lax.fori_loop(0, B, fill, 0)
    rz = NACC // 16
    for z in range(rz // B):
        pltpu.sync_copy(zero_v, acc.at[pl.ds(tid * rz + z * B, B)])
    plsc.subcore_barrier()

    def step(j, _):
        pltpu.sync_copy(ones_v, acc.at[dst_v.at[j]], add=True)
        return 0

    lax.fori_loop(0, NBLK_D, step, 0)
    plsc.subcore_barrier()
    rpt = NACC // 16
    pltpu.sync_copy(acc.at[pl.ds(tid * rpt, rpt)],
                    out.at[cid, pl.ds(tid * rpt, rpt)])


def _make_agg(C, W, pipelined=True):
    wpr = W // 16

    @functools.partial(
        pl.kernel,
        out_type=jax.ShapeDtypeStruct((C * NACC, W), F32),
        mesh=_mesh,
        compiler_params=_sc_params,
        scratch_types=[
            pltpu.VMEM((NBLK, B), jnp.int32),
            pltpu.VMEM((NBLK, B), jnp.int32),
            pltpu.VMEM((B, W), F32),
            pltpu.VMEM((B, W), F32),
            pltpu.VMEM((B, W), F32),
            pltpu.VMEM((B, W), F32),
            pltpu.VMEM((B, W), F32),
            pltpu.VMEM_SHARED((NACC, W), F32),
            pltpu.SemaphoreType.DMA,
            pltpu.SemaphoreType.DMA,
            pltpu.SemaphoreType.DMA,
            pltpu.SemaphoreType.DMA,
            pltpu.SemaphoreType.DMA,
        ],
    )
    def agg_kernel(srcadj, dst3, h, out, adj_v, dst_v, r0, r1, r2, r3, zv,
                   acc, g0, g1, g2, g3, ssc):
        bufs = (r0, r1, r2, r3)
        gsem = (g0, g1, g2, g3)
        cid = lax.axis_index("c")
        tid = lax.axis_index("s")
        pltpu.sync_copy(dst3.at[tid], dst_v)

        def zfill(i, _):
            zv[i // wpr, pl.ds((i % wpr) * 16, 16)] = jnp.zeros((16,), F32)
            return 0

        lax.fori_loop(0, B * wpr, zfill, 0)

        for kc in range(C // 2):
            chunk = kc * 2 + cid
            rz = NACC // 16
            for z in range(rz // B):
                pltpu.sync_copy(zv, acc.at[pl.ds(tid * rz + z * B, B)])

            pltpu.sync_copy(srcadj.at[chunk, tid], adj_v)
            plsc.subcore_barrier()

            if not pipelined:
                def sstep(j, _):
                    pltpu.async_copy(h.at[adj_v.at[j]], r0, g0).wait()
                    pltpu.sync_copy(r0, acc.at[dst_v.at[j]], add=True)
                    return 0

                lax.fori_loop(0, NBLK, sstep, 0)
                plsc.subcore_barrier()
                rpt = NACC // 16
                pltpu.sync_copy(acc.at[pl.ds(tid * rpt, rpt)],
                                out.at[pl.ds(chunk * NACC + tid * rpt, rpt)])
                plsc.subcore_barrier()
                continue

            for b in range(4):
                pltpu.async_copy(h.at[adj_v.at[b]], bufs[b], gsem[b])

            def step(i, _):
                for b in range(4):
                    j = 4 * i + b
                    pltpu.make_async_copy(h.at[adj_v.at[j]], bufs[b],
                                          gsem[b]).wait()
                    pltpu.sync_copy(bufs[b], acc.at[dst_v.at[j]], add=True)

                    @pl.when(j + 4 < NBLK)
                    def _(b=b, j=j):
                        pltpu.async_copy(h.at[adj_v.at[j + 4]], bufs[b],
                                         gsem[b])
                return 0

            lax.fori_loop(0, NBLK // 4, step, 0)
            plsc.subcore_barrier()
            rpt = NACC // 16
            pltpu.sync_copy(acc.at[pl.ds(tid * rpt, rpt)],
                            out.at[pl.ds(chunk * NACC + tid * rpt, rpt)])
            plsc.subcore_barrier()

    return agg_kernel


_agg_x = _make_agg(4, 64)
_agg_h = _make_agg(8, 64)
_agg_y = _make_agg(2, 16)


_DN = (((1,), (1,)), ((), ()))


def _make_tcA(C_in, W_in, dout):
    din = C_in * W_in
    C_out = dout // 64

    def body(agg_ref, degp_ref, h_ref, wl_ref, wr_ref, b_ref,
             z_ref, s_ref, q_ref):
        deg = degp_ref[0, :, 0:1] + degp_ref[1, :, 0:1]
        inv = 1.0 / jnp.maximum(deg, 1.0)
        z = jnp.zeros((BN, dout), F32)
        for c in range(C_in):
            a = agg_ref[c] * inv
            z = z + lax.dot_general(a, wl_ref[:, c * W_in:(c + 1) * W_in],
                                    _DN, preferred_element_type=F32)
            z = z + lax.dot_general(h_ref[c], wr_ref[:, c * W_in:(c + 1) * W_in],
                                    _DN, preferred_element_type=F32)
        z = z + b_ref[...]
        for c in range(C_out):
            z_ref[c] = z[:, c * 64:(c + 1) * 64]
        sp = jnp.sum(z, axis=0, keepdims=True)
        qp = jnp.sum(z * z, axis=0, keepdims=True)

        @pl.when(pl.program_id(0) == 0)
        def _():
            s_ref[...] = sp
            q_ref[...] = qp

        @pl.when(pl.program_id(0) > 0)
        def _():
            s_ref[...] += sp
            q_ref[...] += qp

    return pl.pallas_call(
        body,
        grid=(GRID,),
        in_specs=[
            pl.BlockSpec((C_in, BN, W_in), lambda i: (0, i, 0)),
            pl.BlockSpec((2, BN, 16), lambda i: (0, i, 0)),
            pl.BlockSpec((C_in, BN, W_in), lambda i: (0, i, 0)),
            pl.BlockSpec((dout, din), lambda i: (0, 0)),
            pl.BlockSpec((dout, din), lambda i: (0, 0)),
            pl.BlockSpec((1, dout), lambda i: (0, 0)),
        ],
        out_specs=[
            pl.BlockSpec((C_out, BN, 64), lambda i: (0, i, 0)),
            pl.BlockSpec((1, dout), lambda i: (0, 0)),
            pl.BlockSpec((1, dout), lambda i: (0, 0)),
        ],
        out_shape=[
            jax.ShapeDtypeStruct((C_out, N, 64), F32),
            jax.ShapeDtypeStruct((1, dout), F32),
            jax.ShapeDtypeStruct((1, dout), F32),
        ],
    )


_tcA0 = _make_tcA(4, 64, 512)
_tcA1 = _make_tcA(8, 64, 512)


def _bn_coefs(s_ref, q_ref, g_ref, be_ref):
    mu = s_ref[...] * (1.0 / N)
    var = q_ref[...] * (1.0 / N) - mu * mu
    scale = g_ref[...] * lax.rsqrt(var + EPS)
    shift = be_ref[...] - mu * scale
    return scale, shift


def _make_tcB():
    def body(z_ref, s_ref, q_ref, g_ref, be_ref, h_ref):
        scale, shift = _bn_coefs(s_ref, q_ref, g_ref, be_ref)
        for c in range(8):
            hz = (z_ref[c] * scale[:, c * 64:(c + 1) * 64]
                  + shift[:, c * 64:(c + 1) * 64])
            h_ref[c] = jnp.maximum(hz, 0.0)

    return pl.pallas_call(
        body,
        grid=(GRID,),
        in_specs=[
            pl.BlockSpec((8, BN, 64), lambda i: (0, i, 0)),
            pl.BlockSpec((1, 512), lambda i: (0, 0)),
            pl.BlockSpec((1, 512), lambda i: (0, 0)),
            pl.BlockSpec((1, 512), lambda i: (0, 0)),
            pl.BlockSpec((1, 512), lambda i: (0, 0)),
        ],
        out_specs=[pl.BlockSpec((8, BN, 64), lambda i: (0, i, 0))],
        out_shape=[jax.ShapeDtypeStruct((8, N, 64), F32)],
    )


_tcB = _make_tcB()


def _make_tcBp():
    def body(z_ref, s_ref, q_ref, g_ref, be_ref, wl2_ref, h_ref, y_ref):
        scale, shift = _bn_coefs(s_ref, q_ref, g_ref, be_ref)
        y = jnp.zeros((BN, 32), F32)
        for c in range(8):
            hz = (z_ref[c] * scale[:, c * 64:(c + 1) * 64]
                  + shift[:, c * 64:(c + 1) * 64])
            hc = jnp.maximum(hz, 0.0)
            h_ref[c] = hc
            y = y + lax.dot_general(hc, wl2_ref[:, c * 64:(c + 1) * 64],
                                    _DN, preferred_element_type=F32)
        y_ref[0] = y[:, 0:16]
        y_ref[1] = y[:, 16:32]

    return pl.pallas_call(
        body,
        grid=(GRID,),
        in_specs=[
            pl.BlockSpec((8, BN, 64), lambda i: (0, i, 0)),
            pl.BlockSpec((1, 512), lambda i: (0, 0)),
            pl.BlockSpec((1, 512), lambda i: (0, 0)),
            pl.BlockSpec((1, 512), lambda i: (0, 0)),
            pl.BlockSpec((1, 512), lambda i: (0, 0)),
            pl.BlockSpec((32, 512), lambda i: (0, 0)),
        ],
        out_specs=[
            pl.BlockSpec((8, BN, 64), lambda i: (0, i, 0)),
            pl.BlockSpec((2, BN, 16), lambda i: (0, i, 0)),
        ],
        out_shape=[
            jax.ShapeDtypeStruct((8, N, 64), F32),
            jax.ShapeDtypeStruct((2, N, 16), F32),
        ],
    )


_tcBp = _make_tcBp()


def _make_tcA2():
    def body(agg_ref, degp_ref, h_ref, wr_ref, b_ref, z_ref, s_ref, q_ref):
        deg = degp_ref[0, :, 0:1] + degp_ref[1, :, 0:1]
        inv = 1.0 / jnp.maximum(deg, 1.0)
        mean = jnp.concatenate([agg_ref[0], agg_ref[1]], axis=1) * inv
        z = mean + b_ref[...]
        for c in range(8):
            z = z + lax.dot_general(h_ref[c], wr_ref[:, c * 64:(c + 1) * 64],
                                    _DN, preferred_element_type=F32)
        z_ref[...] = z
        sp = jnp.sum(z, axis=0, keepdims=True)
        qp = jnp.sum(z * z, axis=0, keepdims=True)

        @pl.when(pl.program_id(0) == 0)
        def _():
            s_ref[...] = sp
            q_ref[...] = qp

        @pl.when(pl.program_id(0) > 0)
        def _():
            s_ref[...] += sp
            q_ref[...] += qp

    return pl.pallas_call(
        body,
        grid=(GRID,),
        in_specs=[
            pl.BlockSpec((2, BN, 16), lambda i: (0, i, 0)),
            pl.BlockSpec((2, BN, 16), lambda i: (0, i, 0)),
            pl.BlockSpec((8, BN, 64), lambda i: (0, i, 0)),
            pl.BlockSpec((32, 512), lambda i: (0, 0)),
            pl.BlockSpec((1, 32), lambda i: (0, 0)),
        ],
        out_specs=[
            pl.BlockSpec((BN, 32), lambda i: (i, 0)),
            pl.BlockSpec((1, 32), lambda i: (0, 0)),
            pl.BlockSpec((1, 32), lambda i: (0, 0)),
        ],
        out_shape=[
            jax.ShapeDtypeStruct((N, 32), F32),
            jax.ShapeDtypeStruct((1, 32), F32),
            jax.ShapeDtypeStruct((1, 32), F32),
        ],
    )


_tcA2 = _make_tcA2()


def _make_tcC():
    def body(z_ref, s_ref, q_ref, g_ref, be_ref, hw_ref, hb_ref, o_ref):
        scale, shift = _bn_coefs(s_ref, q_ref, g_ref, be_ref)
        h3 = jnp.maximum(z_ref[...] * scale + shift, 0.0)
        o_ref[...] = (jnp.sum(h3 * hw_ref[...], axis=1, keepdims=True)
                      + hb_ref[0, 0])

    return pl.pallas_call(
        body,
        grid=(GRID,),
        in_specs=[
            pl.BlockSpec((BN, 32), lambda i: (i, 0)),
            pl.BlockSpec((1, 32), lambda i: (0, 0)),
            pl.BlockSpec((1, 32), lambda i: (0, 0)),
            pl.BlockSpec((1, 32), lambda i: (0, 0)),
            pl.BlockSpec((1, 32), lambda i: (0, 0)),
            pl.BlockSpec((1, 32), lambda i: (0, 0)),
            pl.BlockSpec((1, 1), lambda i: (0, 0)),
        ],
        out_specs=[pl.BlockSpec((BN, 1), lambda i: (i, 0))],
        out_shape=[jax.ShapeDtypeStruct((N, 1), F32)],
    )


_tcC = _make_tcC()


def kernel(x, edge_index, Wl0, Wr0, b0, gamma0, beta0, Wl1, Wr1, b1, gamma1,
           beta1, Wl2, Wr2, b2, gamma2, beta2, headW, headb):
    src = edge_index[0]
    dst = edge_index[1]
    pad = EP - E
    src3 = jnp.concatenate([src, jnp.zeros((pad,), jnp.int32)]).reshape(
        16, NBLK, B)
    dst3 = jnp.concatenate([dst, jnp.full((pad,), N, jnp.int32)]).reshape(
        16, NBLK, B)

    def _srcadj(C):
        offs = (jnp.arange(C, dtype=jnp.int32) * N)[:, None, None, None]
        return src3[None] + offs
    x_cm = x.reshape(N, 4, 64).transpose(1, 0, 2)

    degp = _deg_kernel(dst3)

    agg0 = _agg_x(_srcadj(4), dst3, x_cm.reshape(4 * N, 64))
    z0, s0, q0 = _tcA0(agg0.reshape(4, NACC, 64), degp, x_cm, Wl0, Wr0,
                       b0.reshape(1, 512))
    (h1,) = _tcB(z0, s0, q0, gamma0.reshape(1, 512), beta0.reshape(1, 512))

    agg1 = _agg_h(_srcadj(8), dst3, h1.reshape(8 * N, 64))
    z1, s1, q1 = _tcA1(agg1.reshape(8, NACC, 64), degp, h1, Wl1, Wr1,
                       b1.reshape(1, 512))
    h2, y2 = _tcBp(z1, s1, q1, gamma1.reshape(1, 512), beta1.reshape(1, 512),
                   Wl2)

    agg2 = _agg_y(_srcadj(2), dst3, y2.reshape(2 * N, 16))
    z2, s2, q2 = _tcA2(agg2.reshape(2, NACC, 16), degp, h2, Wr2,
                       b2.reshape(1, 32))
    (out,) = _tcC(z2, s2, q2, gamma2.reshape(1, 32), beta2.reshape(1, 32),
                  headW, headb.reshape(1, 1))
    return out
